# baseline (speedup 1.0000x reference)
import functools

import jax
import jax.numpy as jnp
from jax import lax
from jax.experimental import pallas as pl
from jax.experimental.pallas import tpu as pltpu
from jax.experimental.pallas import tpu_sc as plsc

B = 16384
V = 100000
D = 64

NC = 2
NS = 16
NW = NC * NS

VBLK = 2048
NVB = -(-V // VBLK)
VP = NVB * VBLK
BPW = B // NW
IDX_CHUNK = 128
NCH = BPW // IDX_CHUNK
NSLOT = 13


def _prep_body(bagT, embT, eye, out):
  t = (((0,), (0,)), ((), ()))
  hp = lax.Precision.HIGHEST
  out[:, 0:D] = lax.dot_general(bagT[...], eye[...], t, precision=hp,
                                preferred_element_type=jnp.float32)
  out[:, D:2 * D] = lax.dot_general(embT[...], eye[...], t, precision=hp,
                                    preferred_element_type=jnp.float32)


def _prep(bagT, embT):
  eye = jnp.eye(D, dtype=jnp.float32)
  return pl.pallas_call(
      _prep_body,
      grid=(NVB,),
      in_specs=[
          pl.BlockSpec((D, VBLK), lambda i: (0, i)),
          pl.BlockSpec((D, VBLK), lambda i: (0, i)),
          pl.BlockSpec((D, D), lambda i: (0, 0)),
      ],
      out_specs=pl.BlockSpec((VBLK, 2 * D), lambda i: (i, 0)),
      out_shape=jax.ShapeDtypeStruct((VP, 2 * D), jnp.float32),
  )(bagT, embT, eye)


def _sc_gather_body(fused, idx_flat, out, *scratch):
  idx_bufs = scratch[:NSLOT]
  rows_v, sem = scratch[NSLOT:]
  wid = lax.axis_index("s") * NC + lax.axis_index("c")
  base = wid * BPW
  for j in range(NSLOT):
    pltpu.sync_copy(idx_flat.at[pl.ds(j * B + base, BPW)], idx_bufs[j])
  for j in range(NSLOT):
    cps = [
        pltpu.async_copy(
            fused.at[idx_bufs[j].at[pl.ds(c * IDX_CHUNK, IDX_CHUNK)]],
            rows_v.at[pl.ds(c * IDX_CHUNK, IDX_CHUNK)], sem)
        for c in range(NCH)
    ]
    for cp in cps:
      cp.wait()
    pltpu.sync_copy(rows_v, out.at[j, pl.ds(base, BPW)])


def _sc_gather(fused, idx_flat):
  mesh = plsc.VectorSubcoreMesh(core_axis_name="c", subcore_axis_name="s")
  fn = pl.kernel(
      _sc_gather_body,
      out_type=jax.ShapeDtypeStruct((NSLOT, B, 2 * D), jnp.float32),
      mesh=mesh,
      scratch_types=(
          [pltpu.VMEM((BPW,), jnp.int32) for _ in range(NSLOT)]
          + [pltpu.VMEM((BPW, 2 * D), jnp.float32), pltpu.SemaphoreType.DMA]
      ),
      compiler_params=pltpu.CompilerParams(use_tc_tiling_on_sc=True),
  )
  return fn(fused, idx_flat)


R = 1024


def _mlp_body(slots, info, w1s, w1c, b1, w2, b2, w3, b3, out):
  hp = lax.Precision.HIGHEST
  h = jnp.dot(info[...], w1c[...], precision=hp,
              preferred_element_type=jnp.float32)
  for j in range(NSLOT):
    h = h + jnp.dot(slots[j], w1s[j], precision=hp,
                    preferred_element_type=jnp.float32)
  h = jnp.maximum(h + b1[...], 0.0)
  h = jnp.maximum(jnp.dot(h, w2[...], precision=hp,
                          preferred_element_type=jnp.float32) + b2[...], 0.0)
  out[...] = jnp.dot(h, w3[...], precision=hp,
                     preferred_element_type=jnp.float32) + b3[...]


def _mlp(slots, info, w1s, w1c, b1, w2, b2, w3, b3):
  full = lambda shape: pl.BlockSpec(shape, lambda i: (0,) * len(shape))
  return pl.pallas_call(
      _mlp_body,
      grid=(B // R,),
      in_specs=[
          pl.BlockSpec((NSLOT, R, 2 * D), lambda i: (0, i, 0)),
          pl.BlockSpec((R, 8), lambda i: (i, 0)),
          full((NSLOT, 2 * D, D)),
          full((8, D)),
          full((1, D)),
          full((D, 32)),
          full((1, 32)),
          full((32, 1)),
          full((1, 1)),
      ],
      out_specs=pl.BlockSpec((R, 1), lambda i: (i, 0)),
      out_shape=jax.ShapeDtypeStruct((B, 1), jnp.float32),
  )(slots, info, w1s, w1c, b1, w2, b2, w3, b3)


def kernel(offensive_players, defensive_players, shooting_player,
           assisting_player, defending_player, is_putback, is_and1,
           is_freethrow, is_turnover, is_steal, shot_distance,
           emb_table, bag_table, W1, b1, W2, b2, W3, b3):
  fused = _prep(bag_table.astype(jnp.float32).T,
                emb_table.astype(jnp.float32).T)

  idx_flat = jnp.concatenate([
      offensive_players.astype(jnp.int32).T.reshape(-1),
      defensive_players.astype(jnp.int32).T.reshape(-1),
      shooting_player.astype(jnp.int32),
      assisting_player.astype(jnp.int32),
      defending_player.astype(jnp.int32),
  ])
  slots = _sc_gather(fused, idx_flat)

  info = jnp.stack(
      [is_putback, is_and1, is_freethrow, is_turnover, is_steal,
       shot_distance], axis=1)
  info = jnp.pad(info, ((0, 0), (0, 2)))

  z = jnp.zeros((D, D), jnp.float32)
  bag_blk = lambda w: jnp.concatenate([w, z], axis=0)
  emb_blk = lambda w: jnp.concatenate([z, w], axis=0)
  w1s = jnp.stack([bag_blk(W1[0:D])] * 5 + [bag_blk(W1[D:2 * D])] * 5
                  + [emb_blk(W1[2 * D:3 * D]), emb_blk(W1[3 * D:4 * D]),
                     emb_blk(W1[4 * D:5 * D])])
  w1c = jnp.pad(W1[5 * D:5 * D + 6], ((0, 2), (0, 0)))

  return _mlp(slots, info, w1s, w1c,
              b1.reshape(1, D), W2, b2.reshape(1, 32),
              W3, b3.reshape(1, 1))


# speedup vs baseline: 1.6406x; 1.6406x over previous
import functools

import jax
import jax.numpy as jnp
from jax import lax
from jax.experimental import pallas as pl
from jax.experimental.pallas import tpu as pltpu
from jax.experimental.pallas import tpu_sc as plsc

B = 16384
V = 100000
D = 64

NC = 2
NS = 16
NW = NC * NS

VBLK = 2048
NVB = -(-V // VBLK)
VP = NVB * VBLK
BPW = B // NW
IDX_CHUNK = 128
NCH = BPW // IDX_CHUNK
NSLOT = 13


def _prep_body(bagT, embT, out):
  out[:, 0:D] = bagT[...].T
  out[:, D:2 * D] = embT[...].T


def _prep(bagT, embT):
  return pl.pallas_call(
      _prep_body,
      grid=(NVB,),
      in_specs=[
          pl.BlockSpec((D, VBLK), lambda i: (0, i)),
          pl.BlockSpec((D, VBLK), lambda i: (0, i)),
      ],
      out_specs=pl.BlockSpec((VBLK, 2 * D), lambda i: (i, 0)),
      out_shape=jax.ShapeDtypeStruct((VP, 2 * D), jnp.float32),
  )(bagT, embT)


def _sc_gather_body(fused, idx_flat, out, *scratch):
  idx_bufs = scratch[:NSLOT]
  rows_v, sem = scratch[NSLOT:]
  wid = lax.axis_index("s") * NC + lax.axis_index("c")
  base = wid * BPW
  for j in range(NSLOT):
    pltpu.sync_copy(idx_flat.at[pl.ds(j * B + base, BPW)], idx_bufs[j])
  for j in range(NSLOT):
    cps = [
        pltpu.async_copy(
            fused.at[idx_bufs[j].at[pl.ds(c * IDX_CHUNK, IDX_CHUNK)]],
            rows_v.at[pl.ds(c * IDX_CHUNK, IDX_CHUNK)], sem)
        for c in range(NCH)
    ]
    for cp in cps:
      cp.wait()
    pltpu.sync_copy(rows_v, out.at[j, pl.ds(base, BPW)])


def _sc_gather(fused, idx_flat):
  mesh = plsc.VectorSubcoreMesh(core_axis_name="c", subcore_axis_name="s")
  fn = pl.kernel(
      _sc_gather_body,
      out_type=jax.ShapeDtypeStruct((NSLOT, B, 2 * D), jnp.float32),
      mesh=mesh,
      scratch_types=(
          [pltpu.VMEM((BPW,), jnp.int32) for _ in range(NSLOT)]
          + [pltpu.VMEM((BPW, 2 * D), jnp.float32), pltpu.SemaphoreType.DMA]
      ),
      compiler_params=pltpu.CompilerParams(use_tc_tiling_on_sc=True),
  )
  return fn(fused, idx_flat)


R = 1024


def _mlp_body(slots, info, w1s, w1c, b1, w2, b2, w3, b3, out):
  h = jnp.dot(info[...], w1c[...], preferred_element_type=jnp.float32)
  off_p = slots[0] + slots[1] + slots[2] + slots[3] + slots[4]
  def_p = slots[5] + slots[6] + slots[7] + slots[8] + slots[9]
  h = h + jnp.dot(off_p, w1s[0], preferred_element_type=jnp.float32)
  h = h + jnp.dot(def_p, w1s[5], preferred_element_type=jnp.float32)
  for j in range(10, NSLOT):
    h = h + jnp.dot(slots[j], w1s[j], preferred_element_type=jnp.float32)
  h = jnp.maximum(h + b1[...], 0.0)
  h = jnp.maximum(jnp.dot(h, w2[...], preferred_element_type=jnp.float32)
                  + b2[...], 0.0)
  out[...] = jnp.dot(h, w3[...], preferred_element_type=jnp.float32) + b3[...]


def _mlp(slots, info, w1s, w1c, b1, w2, b2, w3, b3):
  full = lambda shape: pl.BlockSpec(shape, lambda i: (0,) * len(shape))
  return pl.pallas_call(
      _mlp_body,
      grid=(B // R,),
      in_specs=[
          pl.BlockSpec((NSLOT, R, 2 * D), lambda i: (0, i, 0)),
          pl.BlockSpec((R, 8), lambda i: (i, 0)),
          full((NSLOT, 2 * D, D)),
          full((8, D)),
          full((1, D)),
          full((D, 32)),
          full((1, 32)),
          full((32, 1)),
          full((1, 1)),
      ],
      out_specs=pl.BlockSpec((R, 1), lambda i: (i, 0)),
      out_shape=jax.ShapeDtypeStruct((B, 1), jnp.float32),
  )(slots, info, w1s, w1c, b1, w2, b2, w3, b3)


def kernel(offensive_players, defensive_players, shooting_player,
           assisting_player, defending_player, is_putback, is_and1,
           is_freethrow, is_turnover, is_steal, shot_distance,
           emb_table, bag_table, W1, b1, W2, b2, W3, b3):
  fused = _prep(bag_table.astype(jnp.float32).T,
                emb_table.astype(jnp.float32).T)

  idx_flat = jnp.concatenate([
      offensive_players.astype(jnp.int32).T.reshape(-1),
      defensive_players.astype(jnp.int32).T.reshape(-1),
      shooting_player.astype(jnp.int32),
      assisting_player.astype(jnp.int32),
      defending_player.astype(jnp.int32),
  ])
  slots = _sc_gather(fused, idx_flat)

  info = jnp.stack(
      [is_putback, is_and1, is_freethrow, is_turnover, is_steal,
       shot_distance], axis=1)
  info = jnp.pad(info, ((0, 0), (0, 2)))

  z = jnp.zeros((D, D), jnp.float32)
  bag_blk = lambda w: jnp.concatenate([w, z], axis=0)
  emb_blk = lambda w: jnp.concatenate([z, w], axis=0)
  w1s = jnp.stack([bag_blk(W1[0:D])] * 5 + [bag_blk(W1[D:2 * D])] * 5
                  + [emb_blk(W1[2 * D:3 * D]), emb_blk(W1[3 * D:4 * D]),
                     emb_blk(W1[4 * D:5 * D])])
  w1c = jnp.pad(W1[5 * D:5 * D + 6], ((0, 2), (0, 0)))

  return _mlp(slots, info, w1s, w1c,
              b1.reshape(1, D), W2, b2.reshape(1, 32),
              W3, b3.reshape(1, 1))
